# Initial kernel scaffold; baseline (speedup 1.0000x reference)
import jax
import jax.numpy as jnp
from jax.experimental import pallas as pl


def kernel(x, edge_index, W1, b1, W2, b2, lin_W, lin_b):
    raise NotImplementedError("write your pallas kernel here")


# speedup vs baseline: 5.4738x; 5.4738x over previous
import functools

import jax
import jax.numpy as jnp
from jax import lax
from jax.experimental import pallas as pl
from jax.experimental.pallas import tpu as pltpu
from jax.experimental.pallas import tpu_sc as plsc

N_NODES = 10000
IN_DIM = 256
HID_DIM = 512
OUT_DIM = 256
HALF = 128

NC = 2
NS = 16
NW = NC * NS
K = 128
NPAD = 10240
ROWS_PER_TILE = NPAD // NS

_MESH = plsc.VectorSubcoreMesh(core_axis_name="c", subcore_axis_name="s")


def _deg_body(dst_hbm, zeros1_hbm, out_hbm, deg_sp, ones_v, dst_v, sem):
    del sem
    cid = lax.axis_index("c")
    sid = lax.axis_index("s")
    wid = cid * NS + sid
    rowbase = sid * ROWS_PER_TILE
    n_chunks = dst_hbm.shape[1]
    pltpu.sync_copy(zeros1_hbm.at[pl.ds(rowbase, ROWS_PER_TILE)],
                    deg_sp.at[pl.ds(rowbase, ROWS_PER_TILE)])
    for k in range(K // 16):
        ones_v[pl.ds(k * 16, 16)] = jnp.ones((16,), jnp.float32)
    plsc.subcore_barrier()

    def step(j, carry):
        pltpu.sync_copy(dst_hbm.at[wid, j], dst_v)
        pltpu.sync_copy(ones_v, deg_sp.at[dst_v], add=True)
        return carry

    lax.fori_loop(0, n_chunks, step, 0)
    plsc.subcore_barrier()
    pltpu.sync_copy(deg_sp.at[pl.ds(rowbase, ROWS_PER_TILE)],
                    out_hbm.at[cid, pl.ds(rowbase, ROWS_PER_TILE)])


def _scatter_body(table_hbm, src_hbm, dst_hbm, zrows_hbm, out_hbm,
                  acc_sp, src_v, dst_v, rows_v, sem):
    cid = lax.axis_index("c")
    sid = lax.axis_index("s")
    wid = cid * NS + sid
    rowbase = sid * ROWS_PER_TILE
    n_chunks = src_hbm.shape[1]
    pltpu.sync_copy(zrows_hbm.at[pl.ds(rowbase, ROWS_PER_TILE)],
                    acc_sp.at[pl.ds(rowbase, ROWS_PER_TILE)])
    plsc.subcore_barrier()

    def step(j, carry):
        pltpu.sync_copy(src_hbm.at[wid, j], src_v)
        pltpu.sync_copy(dst_hbm.at[wid, j], dst_v)
        pltpu.async_copy(table_hbm.at[src_v], rows_v, sem).wait()
        pltpu.sync_copy(rows_v, acc_sp.at[dst_v], add=True)
        return carry

    lax.fori_loop(0, n_chunks, step, 0)
    plsc.subcore_barrier()
    pltpu.sync_copy(acc_sp.at[pl.ds(rowbase, ROWS_PER_TILE)],
                    out_hbm.at[cid, pl.ds(rowbase, ROWS_PER_TILE)])


def _make_deg_call(n_chunks):
    return pl.kernel(
        _deg_body,
        out_type=jax.ShapeDtypeStruct((NC, NPAD), jnp.float32),
        mesh=_MESH,
        scratch_types=[
            pltpu.VMEM_SHARED((NPAD,), jnp.float32),
            pltpu.VMEM((K,), jnp.float32),
            pltpu.VMEM((K,), jnp.int32),
            pltpu.SemaphoreType.DMA,
        ],
    )


def _make_scatter_call(n_chunks):
    return pl.kernel(
        _scatter_body,
        out_type=jax.ShapeDtypeStruct((NC, NPAD, HALF), jnp.float32),
        mesh=_MESH,
        scratch_types=[
            pltpu.VMEM_SHARED((NPAD, HALF), jnp.float32),
            pltpu.VMEM((K,), jnp.int32),
            pltpu.VMEM((K,), jnp.int32),
            pltpu.VMEM((K, HALF), jnp.float32),
            pltpu.SemaphoreType.DMA,
        ],
    )


def _prep_body(degp_ref, x_ref, dinv_ref, x1a_ref, x1b_ref):
    deg = degp_ref[:, 0:1] + degp_ref[:, 1:2] + 1.0
    dinv = lax.rsqrt(deg)
    dinv_ref[...] = dinv
    d = dinv[:N_NODES]
    x = x_ref[...]
    x1a_ref[...] = x[:, :HALF] * d
    x1b_ref[...] = x[:, HALF:] * d


def _mid_body(s1a_ref, s1b_ref, x1a_ref, x1b_ref, dinv_ref,
              W1_ref, b1_ref, W2_ref, x2a_ref, x2b_ref):
    d = dinv_ref[...]
    a = (s1a_ref[0] + s1a_ref[1] + x1a_ref[...]) * d
    b = (s1b_ref[0] + s1b_ref[1] + x1b_ref[...]) * d
    agg1 = jnp.concatenate([a, b], axis=1)
    h = jnp.dot(agg1, W1_ref[...], preferred_element_type=jnp.float32)
    h = jnp.maximum(h + b1_ref[...], 0.0)
    p = jnp.dot(h, W2_ref[...], preferred_element_type=jnp.float32)
    x2 = p * d
    x2a_ref[...] = x2[:, :HALF]
    x2b_ref[...] = x2[:, HALF:]


def _out_body(s2a_ref, s2b_ref, x2a_ref, x2b_ref, dinv_ref,
              b2_ref, linW_ref, linb_ref, emb_ref, score_ref):
    d = dinv_ref[...]
    a = (s2a_ref[0] + s2a_ref[1] + x2a_ref[...]) * d
    b = (s2b_ref[0] + s2b_ref[1] + x2b_ref[...]) * d
    emb = jnp.concatenate([a, b], axis=1) + b2_ref[...]
    emb_ref[...] = emb
    z = jnp.dot(emb, linW_ref[...], preferred_element_type=jnp.float32)
    score_ref[...] = jax.nn.sigmoid(z + linb_ref[...])


def kernel(x, edge_index, W1, b1, W2, b2, lin_W, lin_b):
    n = x.shape[0]
    assert n == N_NODES
    e = edge_index.shape[1]
    epad = -(-e // (NW * K)) * (NW * K)
    n_chunks = epad // (NW * K)

    src = edge_index[0].astype(jnp.int32)
    dst = edge_index[1].astype(jnp.int32)
    src3 = jnp.concatenate(
        [src, jnp.zeros((epad - e,), jnp.int32)]).reshape(NW, n_chunks, K)
    dst3 = jnp.concatenate(
        [dst, jnp.full((epad - e,), n, jnp.int32)]).reshape(NW, n_chunks, K)
    zeros1 = jnp.zeros((NPAD,), jnp.float32)
    zrows = jnp.zeros((NPAD, HALF), jnp.float32)
    b1_2d = b1.reshape(1, HID_DIM)
    b2_2d = b2.reshape(1, OUT_DIM)
    linb_2d = lin_b.reshape(1, 1)

    degp = _make_deg_call(n_chunks)(dst3, zeros1)
    degp_t = degp.T

    dinv, x1a, x1b = pl.pallas_call(
        _prep_body,
        out_shape=[
            jax.ShapeDtypeStruct((NPAD, 1), jnp.float32),
            jax.ShapeDtypeStruct((n, HALF), jnp.float32),
            jax.ShapeDtypeStruct((n, HALF), jnp.float32),
        ],
    )(degp_t, x)

    scat = _make_scatter_call(n_chunks)

    s1a = scat(x1a, src3, dst3, zrows)
    s1b = scat(x1b, src3, dst3, zrows)

    R = 2000
    grid = (n // R,)
    part_spec = pl.BlockSpec((NC, R, HALF), lambda r: (0, r, 0))
    half_spec = pl.BlockSpec((R, HALF), lambda r: (r, 0))
    dinv_spec = pl.BlockSpec((R, 1), lambda r: (r, 0))
    x2a, x2b = pl.pallas_call(
        _mid_body,
        grid=grid,
        in_specs=[
            part_spec, part_spec, half_spec, half_spec, dinv_spec,
            pl.BlockSpec((IN_DIM, HID_DIM), lambda r: (0, 0)),
            pl.BlockSpec((1, HID_DIM), lambda r: (0, 0)),
            pl.BlockSpec((HID_DIM, OUT_DIM), lambda r: (0, 0)),
        ],
        out_specs=[half_spec, half_spec],
        out_shape=[
            jax.ShapeDtypeStruct((n, HALF), jnp.float32),
            jax.ShapeDtypeStruct((n, HALF), jnp.float32),
        ],
    )(s1a, s1b, x1a, x1b, dinv, W1, b1_2d, W2)

    s2a = scat(x2a, src3, dst3, zrows)
    s2b = scat(x2b, src3, dst3, zrows)

    emb, score = pl.pallas_call(
        _out_body,
        grid=grid,
        in_specs=[
            part_spec, part_spec, half_spec, half_spec, dinv_spec,
            pl.BlockSpec((1, OUT_DIM), lambda r: (0, 0)),
            pl.BlockSpec((OUT_DIM, 1), lambda r: (0, 0)),
            pl.BlockSpec((1, 1), lambda r: (0, 0)),
        ],
        out_specs=[
            pl.BlockSpec((R, OUT_DIM), lambda r: (r, 0)),
            pl.BlockSpec((R, 1), lambda r: (r, 0)),
        ],
        out_shape=[
            jax.ShapeDtypeStruct((n, OUT_DIM), jnp.float32),
            jax.ShapeDtypeStruct((n, 1), jnp.float32),
        ],
    )(s2a, s2b, x2a, x2b, dinv, b2_2d, lin_W, linb_2d)

    return emb, score[:, 0]
